# Initial kernel scaffold; baseline (speedup 1.0000x reference)
import jax
import jax.numpy as jnp
from jax.experimental import pallas as pl


def kernel(h, x, edge_index, mask_ligand, edge_attr, We1, be1, We2, be2, Wg, bg, Wn1, bn1, Wn2, bn2, Wx1, bx1, Wx2):
    raise NotImplementedError("write your pallas kernel here")


# speedup vs baseline: 3.2851x; 3.2851x over previous
import functools

import jax
import jax.numpy as jnp
from jax import lax
from jax.experimental import pallas as pl
from jax.experimental.pallas import tpu as pltpu
from jax.experimental.pallas import tpu_sc as plsc

CUTOFF = 10.0
NGAUSS = 16
CH = 128
NC = 2
NS = 16


def _silu(v):
    return v * jax.nn.sigmoid(v)


def _precompute_tc(h, W1d, W1s):
    N, H = h.shape

    def body(h_ref, wd_ref, ws_ref, hd_ref, hs_ref):
        hb = h_ref[...]
        hd_ref[...] = jnp.dot(hb, wd_ref[...], preferred_element_type=jnp.float32)
        hs_ref[...] = jnp.dot(hb, ws_ref[...], preferred_element_type=jnp.float32)

    return pl.pallas_call(
        body,
        out_shape=(
            jax.ShapeDtypeStruct((N, H), jnp.float32),
            jax.ShapeDtypeStruct((N, H), jnp.float32),
        ),
    )(h, W1d, W1s)


def _gather_sc(hd, hs, xpad, src, dst):
    N, H = hd.shape
    E = src.shape[0]
    NCH = E // CH
    NW = NC * NS
    JMAX = (NCH + NW - 1) // NW
    mesh = plsc.VectorSubcoreMesh(
        core_axis_name="c", subcore_axis_name="s", num_cores=NC, num_subcores=NS)

    @functools.partial(
        pl.kernel,
        out_type=(
            jax.ShapeDtypeStruct((E, H), jnp.float32),
            jax.ShapeDtypeStruct((E, H), jnp.float32),
            jax.ShapeDtypeStruct((E, 128), jnp.float32),
            jax.ShapeDtypeStruct((E, 128), jnp.float32),
        ),
        mesh=mesh,
        scratch_types=[
            pltpu.VMEM((CH,), jnp.int32),
            pltpu.VMEM((CH,), jnp.int32),
            pltpu.VMEM((CH, H), jnp.float32),
            pltpu.VMEM((CH, H), jnp.float32),
            pltpu.VMEM((CH, 128), jnp.float32),
            pltpu.VMEM((CH, 128), jnp.float32),
            pltpu.SemaphoreType.DMA,
        ],
    )
    def k(hd_hbm, hs_hbm, xp_hbm, src_hbm, dst_hbm,
          hd_out, hs_out, xd_out, xs_out,
          sidx, didx, hdbuf, hsbuf, xdbuf, xsbuf, sem):
        wid = lax.axis_index("s") * NC + lax.axis_index("c")

        def body(j, carry):
            c = j * NW + wid

            @pl.when(c < NCH)
            def _():
                base = c * CH
                pltpu.sync_copy(dst_hbm.at[pl.ds(base, CH)], didx)
                pltpu.sync_copy(src_hbm.at[pl.ds(base, CH)], sidx)
                d1 = pltpu.async_copy(hd_hbm.at[didx], hdbuf, sem)
                d2 = pltpu.async_copy(hs_hbm.at[sidx], hsbuf, sem)
                d3 = pltpu.async_copy(xp_hbm.at[didx], xdbuf, sem)
                d4 = pltpu.async_copy(xp_hbm.at[sidx], xsbuf, sem)
                d1.wait()
                d2.wait()
                d3.wait()
                d4.wait()
                pltpu.sync_copy(hdbuf, hd_out.at[pl.ds(base, CH)])
                pltpu.sync_copy(hsbuf, hs_out.at[pl.ds(base, CH)])
                pltpu.sync_copy(xdbuf, xd_out.at[pl.ds(base, CH)])
                pltpu.sync_copy(xsbuf, xs_out.at[pl.ds(base, CH)])

            return carry

        lax.fori_loop(0, JMAX, body, 0)

    return k(hd, hs, xpad, src, dst)


def _edge_tc(HD, HS, XD, XS, ea, W1e, be1, We2, be2, WgT, bg, Wx1, bx1, Wx2T):
    E, H = HD.shape
    EF = ea.shape[1]
    BE = 2000
    G = E // BE
    step = CUTOFF / (NGAUSS - 1)
    coeff = -0.5 / step**2

    def body(hd_ref, hs_ref, xd_ref, xs_ref, ea_ref, w1e_ref, be1_ref,
             we2_ref, be2_ref, wgt_ref, bg_ref, wx1_ref, bx1_ref, wx2t_ref,
             msg_ref, vec_ref):
        rel = xd_ref[...] - xs_ref[...]
        d2 = jnp.sum(rel * rel, axis=1, keepdims=True)
        r = jnp.sqrt(d2 + 1e-8)
        offs = lax.broadcasted_iota(jnp.int32, (1, NGAUSS), 1).astype(jnp.float32) * step
        dfe = jnp.exp(coeff * (r - offs) ** 2)
        ef = jnp.concatenate([dfe, ea_ref[...]], axis=1)
        pre1 = (hd_ref[...] + hs_ref[...]
                + jnp.dot(ef, w1e_ref[...], preferred_element_type=jnp.float32)
                + be1_ref[...])
        m1 = _silu(pre1)
        m = _silu(jnp.dot(m1, we2_ref[...], preferred_element_type=jnp.float32)
                  + be2_ref[...])
        gpre = jnp.sum(m * wgt_ref[...], axis=1, keepdims=True) + bg_ref[...]
        msg_ref[...] = m * jax.nn.sigmoid(gpre)
        t = _silu(jnp.dot(m, wx1_ref[...], preferred_element_type=jnp.float32)
                  + bx1_ref[...])
        coef = jnp.tanh(jnp.sum(t * wx2t_ref[...], axis=1, keepdims=True))
        vec_ref[...] = rel / (r + 1.0) * coef

    full = lambda s0, s1: pl.BlockSpec((s0, s1), lambda i: (0, 0))
    eb = lambda w: pl.BlockSpec((BE, w), lambda i: (i, 0))
    return pl.pallas_call(
        body,
        grid=(G,),
        in_specs=[
            eb(H), eb(H), eb(128), eb(128), eb(EF),
            full(NGAUSS + EF, H), full(1, H), full(H, H), full(1, H),
            full(1, H), full(1, 1), full(H, H), full(1, H), full(1, H),
        ],
        out_specs=[eb(H), eb(128)],
        out_shape=(
            jax.ShapeDtypeStruct((E, H), jnp.float32),
            jax.ShapeDtypeStruct((E, 128), jnp.float32),
        ),
    )(HD, HS, XD, XS, ea, W1e, be1, We2, be2, WgT, bg, Wx1, bx1, Wx2T)


def _scatter_sc(val, dst, N):
    E, W = val.shape
    NCH = E // CH
    NW = NC * NS
    JMAX = (NCH + NW - 1) // NW
    RPT = ((N + NS * 8 - 1) // (NS * 8)) * 8
    NP = RPT * NS
    mesh = plsc.VectorSubcoreMesh(
        core_axis_name="c", subcore_axis_name="s", num_cores=NC, num_subcores=NS)

    @functools.partial(
        pl.kernel,
        out_type=jax.ShapeDtypeStruct((NC * NP, W), jnp.float32),
        mesh=mesh,
        scratch_types=[
            pltpu.VMEM((CH,), jnp.int32),
            pltpu.VMEM((CH, W), jnp.float32),
            pltpu.VMEM_SHARED((NP, W), jnp.float32),
        ],
    )
    def k(val_hbm, dst_hbm, acc_out, idx, vbuf, acc_sp):
        cid = lax.axis_index("c")
        sid = lax.axis_index("s")
        wid = sid * NC + cid
        z = jnp.zeros((16,), jnp.float32)

        def zb(i, carry):
            for l in range(W // 16):
                vbuf[i, pl.ds(l * 16, 16)] = z
            return carry

        lax.fori_loop(0, CH, zb, 0)
        for off in range(0, RPT, CH):
            zh = min(CH, RPT - off)
            pltpu.sync_copy(vbuf.at[pl.ds(0, zh)],
                            acc_sp.at[pl.ds(sid * RPT + off, zh)])
        plsc.subcore_barrier()

        def body(j, carry):
            c = j * NW + wid

            @pl.when(c < NCH)
            def _():
                base = c * CH
                pltpu.sync_copy(dst_hbm.at[pl.ds(base, CH)], idx)
                pltpu.sync_copy(val_hbm.at[pl.ds(base, CH)], vbuf)
                pltpu.sync_copy(vbuf, acc_sp.at[idx], add=True)

            return carry

        lax.fori_loop(0, JMAX, body, 0)
        plsc.subcore_barrier()
        row0 = sid * RPT
        out0 = cid * NP + sid * RPT
        pltpu.sync_copy(acc_sp.at[pl.ds(row0, RPT)], acc_out.at[pl.ds(out0, RPT)])

    return k(val, dst), NP


def _node_tc(aggp, dxp, h, xpad, maskf, Wn1a, Wn1b, bn1, Wn2, bn2, NP):
    N, H = h.shape

    def body(aggp_ref, dxp_ref, h_ref, xp_ref, mk_ref, wa_ref, wb_ref,
             bn1_ref, wn2_ref, bn2_ref, ho_ref, xo_ref):
        agg = aggp_ref[pl.ds(0, N), :] + aggp_ref[pl.ds(NP, N), :]
        hb = h_ref[...]
        nh = _silu(jnp.dot(agg, wa_ref[...], preferred_element_type=jnp.float32)
                   + jnp.dot(hb, wb_ref[...], preferred_element_type=jnp.float32)
                   + bn1_ref[...])
        ho_ref[...] = hb + jnp.dot(nh, wn2_ref[...],
                                   preferred_element_type=jnp.float32) + bn2_ref[...]
        dx = dxp_ref[pl.ds(0, N), :] + dxp_ref[pl.ds(NP, N), :]
        xo_ref[...] = xp_ref[...] + dx * mk_ref[...]

    return pl.pallas_call(
        body,
        out_shape=(
            jax.ShapeDtypeStruct((N, H), jnp.float32),
            jax.ShapeDtypeStruct((N, 128), jnp.float32),
        ),
    )(aggp, dxp, h, xpad, maskf, Wn1a, Wn1b, bn1, Wn2, bn2)


def kernel(h, x, edge_index, mask_ligand, edge_attr,
           We1, be1, We2, be2, Wg, bg, Wn1, bn1, Wn2, bn2, Wx1, bx1, Wx2):
    N, H = h.shape
    E = edge_index.shape[1]
    src = edge_index[0]
    dst = edge_index[1]
    xpad = jnp.pad(x, ((0, 0), (0, 128 - x.shape[1])))

    hd, hs = _precompute_tc(h, We1[:H], We1[H:2 * H])
    HD, HS, XD, XS = _gather_sc(hd, hs, xpad, src, dst)
    msg, vecp = _edge_tc(
        HD, HS, XD, XS, edge_attr,
        We1[2 * H:], be1.reshape(1, H), We2, be2.reshape(1, H),
        Wg.reshape(1, H), bg.reshape(1, 1), Wx1, bx1.reshape(1, H),
        Wx2.reshape(1, H),
    )
    aggp, NP = _scatter_sc(msg, dst, N)
    dxp, _ = _scatter_sc(vecp, dst, N)
    h_out, x_out_pad = _node_tc(
        aggp, dxp, h, xpad, mask_ligand.astype(jnp.float32).reshape(N, 1),
        Wn1[:H], Wn1[H:], bn1.reshape(1, H), Wn2, bn2.reshape(1, H), NP,
    )
    return h_out, x_out_pad[:, :x.shape[1]]
